# baseline (speedup 1.0000x reference)
import functools

import jax
import jax.numpy as jnp
from jax import lax
from jax.experimental import pallas as pl
from jax.experimental.pallas import tpu as pltpu
from jax.experimental.pallas import tpu_sc as plsc

N_NODES = 10000
D = 128
NP = 10240
NC = 2
NS = 16
NW = NC * NS
CH = 128
CPW = 78
XTRA = 4
EPW = CPW * CH
RPT = NP // NS

_mesh = plsc.VectorSubcoreMesh(
    core_axis_name="c", subcore_axis_name="s", num_cores=NC, num_subcores=NS
)


@functools.partial(
    pl.kernel,
    out_type=jax.ShapeDtypeStruct((NC, NP), jnp.float32),
    mesh=_mesh,
    scratch_types=[
        pltpu.VMEM((2, CH), jnp.int32),
        pltpu.VMEM((CH,), jnp.float32),
        pltpu.VMEM((RPT,), jnp.float32),
        pltpu.VMEM_SHARED((NP,), jnp.float32),
        pltpu.SemaphoreType.DMA,
        pltpu.SemaphoreType.DMA,
    ],
)
def _deg_kernel(e_hbm, degp_hbm, idx, ones_v, z_v, deg_sh, semd0, semd1):
    cid = lax.axis_index("c")
    sid = lax.axis_index("s")
    wid = sid * NC + cid
    base = wid * EPW

    def fill_zero(i, carry):
        z_v[pl.ds(i * 16, 16)] = jnp.zeros((16,), jnp.float32)
        return carry

    lax.fori_loop(0, RPT // 16, fill_zero, 0)
    for j in range(CH // 16):
        ones_v[pl.ds(j * 16, 16)] = jnp.ones((16,), jnp.float32)
    pltpu.sync_copy(z_v, deg_sh.at[pl.ds(sid * RPT, RPT)])
    plsc.subcore_barrier()

    pltpu.async_copy(e_hbm.at[1, pl.ds(base, CH)], idx.at[0], semd0)
    pltpu.async_copy(e_hbm.at[1, pl.ds(base + CH, CH)], idx.at[1], semd1)

    def pair(g, carry):
        c0 = 2 * g
        pltpu.make_async_copy(
            e_hbm.at[1, pl.ds(base + c0 * CH, CH)], idx.at[0], semd0
        ).wait()
        pltpu.sync_copy(ones_v, deg_sh.at[idx.at[0]], add=True)

        @pl.when(g < CPW // 2 - 1)
        def _():
            pltpu.async_copy(
                e_hbm.at[1, pl.ds(base + (c0 + 2) * CH, CH)],
                idx.at[0], semd0,
            )

        pltpu.make_async_copy(
            e_hbm.at[1, pl.ds(base + (c0 + 1) * CH, CH)], idx.at[1], semd1
        ).wait()
        pltpu.sync_copy(ones_v, deg_sh.at[idx.at[1]], add=True)

        @pl.when(g < CPW // 2 - 1)
        def _():
            pltpu.async_copy(
                e_hbm.at[1, pl.ds(base + (c0 + 3) * CH, CH)],
                idx.at[1], semd1,
            )

        return carry

    lax.fori_loop(0, CPW // 2, pair, 0)

    @pl.when(wid < XTRA)
    def _():
        pltpu.sync_copy(
            e_hbm.at[1, pl.ds((NW * CPW + wid) * CH, CH)], idx.at[0]
        )
        pltpu.sync_copy(ones_v, deg_sh.at[idx.at[0]], add=True)

    plsc.subcore_barrier()
    pltpu.sync_copy(
        deg_sh.at[pl.ds(sid * RPT, RPT)],
        degp_hbm.at[cid, pl.ds(sid * RPT, RPT)],
    )


@functools.partial(
    pl.kernel,
    out_type=jax.ShapeDtypeStruct((NC, NP, D), jnp.float32),
    mesh=_mesh,
    scratch_types=[
        pltpu.VMEM((EPW + CH, ), jnp.int32),
        pltpu.VMEM((2, CH), jnp.int32),
        pltpu.VMEM((CH, D), jnp.float32),
        pltpu.VMEM((CH, D), jnp.float32),
        pltpu.VMEM_SHARED((NP, D), jnp.float32),
        pltpu.SemaphoreType.DMA,
        pltpu.SemaphoreType.DMA,
        pltpu.SemaphoreType.DMA,
        pltpu.SemaphoreType.DMA,
    ],
)
def _spmm_kernel(y_hbm, e_hbm, acc_hbm, isrc, idst, buf0, buf1,
                 acc_sh, sem0, sem1, semd0, semd1):
    cid = lax.axis_index("c")
    sid = lax.axis_index("s")
    wid = sid * NC + cid
    base = wid * EPW

    zeros16 = jnp.zeros((16,), jnp.float32)

    def fill_zero(i, carry):
        for j in range(D // 16):
            buf0[i, pl.ds(j * 16, 16)] = zeros16
        return carry

    lax.fori_loop(0, CH, fill_zero, 0)

    def zero_acc(k, carry):
        pltpu.sync_copy(buf0, acc_sh.at[pl.ds(sid * RPT + k * CH, CH)])
        return carry

    lax.fori_loop(0, RPT // CH, zero_acc, 0)
    pltpu.sync_copy(e_hbm.at[1, pl.ds(base, EPW)], isrc.at[pl.ds(0, EPW)])
    plsc.subcore_barrier()

    @pl.when(wid < XTRA)
    def _():
        xoff = (NW * CPW + wid) * CH
        pltpu.sync_copy(
            e_hbm.at[1, pl.ds(xoff, CH)], isrc.at[pl.ds(EPW, CH)]
        )
        pltpu.sync_copy(e_hbm.at[0, pl.ds(xoff, CH)], idst.at[0])
        pltpu.async_copy(y_hbm.at[isrc.at[pl.ds(EPW, CH)]], buf0, sem0)
        pltpu.make_async_copy(
            y_hbm.at[isrc.at[pl.ds(EPW, CH)]], buf0, sem0
        ).wait()
        pltpu.sync_copy(buf0, acc_sh.at[idst.at[0]], add=True)

    pltpu.async_copy(e_hbm.at[0, pl.ds(base, CH)], idst.at[0], semd0)
    pltpu.async_copy(e_hbm.at[0, pl.ds(base + CH, CH)], idst.at[1], semd1)
    pltpu.async_copy(y_hbm.at[isrc.at[pl.ds(0, CH)]], buf0, sem0)
    pltpu.async_copy(y_hbm.at[isrc.at[pl.ds(CH, CH)]], buf1, sem1)

    def pair(g, carry):
        c0 = 2 * g
        pltpu.make_async_copy(
            y_hbm.at[isrc.at[pl.ds(c0 * CH, CH)]], buf0, sem0
        ).wait()
        pltpu.make_async_copy(
            e_hbm.at[0, pl.ds(base + c0 * CH, CH)], idst.at[0], semd0
        ).wait()
        pltpu.sync_copy(buf0, acc_sh.at[idst.at[0]], add=True)

        @pl.when(g < CPW // 2 - 1)
        def _():
            pltpu.async_copy(
                e_hbm.at[0, pl.ds(base + (c0 + 2) * CH, CH)],
                idst.at[0], semd0,
            )
            pltpu.async_copy(
                y_hbm.at[isrc.at[pl.ds((c0 + 2) * CH, CH)]], buf0, sem0
            )

        pltpu.make_async_copy(
            y_hbm.at[isrc.at[pl.ds((c0 + 1) * CH, CH)]], buf1, sem1
        ).wait()
        pltpu.make_async_copy(
            e_hbm.at[0, pl.ds(base + (c0 + 1) * CH, CH)], idst.at[1], semd1
        ).wait()
        pltpu.sync_copy(buf1, acc_sh.at[idst.at[1]], add=True)

        @pl.when(g < CPW // 2 - 1)
        def _():
            pltpu.async_copy(
                e_hbm.at[0, pl.ds(base + (c0 + 3) * CH, CH)],
                idst.at[1], semd1,
            )
            pltpu.async_copy(
                y_hbm.at[isrc.at[pl.ds((c0 + 3) * CH, CH)]], buf1, sem1
            )

        return carry

    lax.fori_loop(0, CPW // 2, pair, 0)
    plsc.subcore_barrier()
    pltpu.sync_copy(
        acc_sh.at[pl.ds(sid * RPT, RPT)],
        acc_hbm.at[cid, pl.ds(sid * RPT, RPT)],
    )


def _t1_body(da_ref, db_ref, x_ref, dis_ref, dis2_ref, y_ref):
    deg = da_ref[...] + db_ref[...] + 1.0
    dis = lax.rsqrt(deg)
    dis_ref[...] = dis
    dis2_ref[...] = dis * dis
    y_ref[: N_NODES, :] = dis[: N_NODES, :] * x_ref[...]
    y_ref[N_NODES :, :] = jnp.zeros((NP - N_NODES, D), jnp.float32)


_t1 = pl.pallas_call(
    _t1_body,
    out_shape=(
        jax.ShapeDtypeStruct((NP, 1), jnp.float32),
        jax.ShapeDtypeStruct((NP, 1), jnp.float32),
        jax.ShapeDtypeStruct((NP, D), jnp.float32),
    ),
)


def _combine_body(s_ref, acc_ref, y_ref, o_ref):
    o_ref[...] = s_ref[...] * (acc_ref[0] + acc_ref[1] + y_ref[...])


_combine = pl.pallas_call(
    _combine_body,
    out_shape=jax.ShapeDtypeStruct((NP, D), jnp.float32),
)


def _combine_final_body(s_ref, acc_ref, y_ref, o_ref):
    n = N_NODES
    o_ref[...] = s_ref[:n, :] * (
        acc_ref[0, :n, :] + acc_ref[1, :n, :] + y_ref[:n, :]
    )


_combine_final = pl.pallas_call(
    _combine_final_body,
    out_shape=jax.ShapeDtypeStruct((N_NODES, D), jnp.float32),
)


def kernel(edge_index, x):
    ei = edge_index.astype(jnp.int32)
    degp = _deg_kernel(ei)
    da = degp[0].reshape(NP, 1)
    db = degp[1].reshape(NP, 1)
    dis, dis2, y1 = _t1(da, db, x)
    acc1 = _spmm_kernel(y1, ei)
    y2 = _combine(dis2, acc1, y1)
    acc2 = _spmm_kernel(y2, ei)
    return _combine_final(dis, acc2, y2)


# speedup vs baseline: 1.0352x; 1.0352x over previous
import functools

import jax
import jax.numpy as jnp
from jax import lax
from jax.experimental import pallas as pl
from jax.experimental.pallas import tpu as pltpu
from jax.experimental.pallas import tpu_sc as plsc

N_NODES = 10000
D = 128
NP = 10240
NC = 2
NS = 16
NW = NC * NS
CH = 128
CPW = 78
XTRA = 4
EPW = CPW * CH
RPT = NP // NS

_mesh = plsc.VectorSubcoreMesh(
    core_axis_name="c", subcore_axis_name="s", num_cores=NC, num_subcores=NS
)


@functools.partial(
    pl.kernel,
    out_type=jax.ShapeDtypeStruct((NC, NP), jnp.float32),
    mesh=_mesh,
    scratch_types=[
        pltpu.VMEM((4, CH), jnp.int32),
        pltpu.VMEM((CH,), jnp.float32),
        pltpu.VMEM((RPT,), jnp.float32),
        pltpu.VMEM_SHARED((NP,), jnp.float32),
        pltpu.SemaphoreType.DMA,
        pltpu.SemaphoreType.DMA,
        pltpu.SemaphoreType.DMA,
        pltpu.SemaphoreType.DMA,
    ],
)
def _deg_kernel(e_hbm, degp_hbm, idx, ones_v, z_v, deg_sh,
                semd0, semd1, semd2, semd3):
    cid = lax.axis_index("c")
    sid = lax.axis_index("s")
    wid = sid * NC + cid
    base = wid * EPW
    semd = (semd0, semd1, semd2, semd3)

    def fill_zero(i, carry):
        z_v[pl.ds(i * 16, 16)] = jnp.zeros((16,), jnp.float32)
        return carry

    lax.fori_loop(0, RPT // 16, fill_zero, 0)
    for j in range(CH // 16):
        ones_v[pl.ds(j * 16, 16)] = jnp.ones((16,), jnp.float32)
    pltpu.sync_copy(z_v, deg_sh.at[pl.ds(sid * RPT, RPT)])
    plsc.subcore_barrier()

    for j in range(4):
        pltpu.async_copy(
            e_hbm.at[1, pl.ds(base + j * CH, CH)], idx.at[j], semd[j]
        )

    def quad(h, carry):
        c0 = 4 * h
        for j in range(4):
            pltpu.make_async_copy(
                e_hbm.at[1, pl.ds(base + (c0 + j) * CH, CH)],
                idx.at[j], semd[j],
            ).wait()
            pltpu.sync_copy(ones_v, deg_sh.at[idx.at[j]], add=True)

            @pl.when(c0 + j + 4 < CPW)
            def _():
                pltpu.async_copy(
                    e_hbm.at[1, pl.ds(base + (c0 + j + 4) * CH, CH)],
                    idx.at[j], semd[j],
                )

        return carry

    lax.fori_loop(0, CPW // 4, quad, 0)
    for j in range(CPW - 4 * (CPW // 4)):
        pltpu.make_async_copy(
            e_hbm.at[1, pl.ds(base + (4 * (CPW // 4) + j) * CH, CH)],
            idx.at[j], semd[j],
        ).wait()
        pltpu.sync_copy(ones_v, deg_sh.at[idx.at[j]], add=True)

    @pl.when(wid < XTRA)
    def _():
        pltpu.sync_copy(
            e_hbm.at[1, pl.ds((NW * CPW + wid) * CH, CH)], idx.at[0]
        )
        pltpu.sync_copy(ones_v, deg_sh.at[idx.at[0]], add=True)

    plsc.subcore_barrier()
    pltpu.sync_copy(
        deg_sh.at[pl.ds(sid * RPT, RPT)],
        degp_hbm.at[cid, pl.ds(sid * RPT, RPT)],
    )


@functools.partial(
    pl.kernel,
    out_type=jax.ShapeDtypeStruct((NC, NP, D), jnp.float32),
    mesh=_mesh,
    scratch_types=[
        pltpu.VMEM((EPW + CH, ), jnp.int32),
        pltpu.VMEM((2, CH), jnp.int32),
        pltpu.VMEM((CH, D), jnp.float32),
        pltpu.VMEM((CH, D), jnp.float32),
        pltpu.VMEM_SHARED((NP, D), jnp.float32),
        pltpu.SemaphoreType.DMA,
        pltpu.SemaphoreType.DMA,
        pltpu.SemaphoreType.DMA,
        pltpu.SemaphoreType.DMA,
    ],
)
def _spmm_kernel(y_hbm, e_hbm, acc_hbm, isrc, idst, buf0, buf1,
                 acc_sh, sem0, sem1, semd0, semd1):
    cid = lax.axis_index("c")
    sid = lax.axis_index("s")
    wid = sid * NC + cid
    base = wid * EPW

    zeros16 = jnp.zeros((16,), jnp.float32)

    def fill_zero(i, carry):
        for j in range(D // 16):
            buf0[i, pl.ds(j * 16, 16)] = zeros16
        return carry

    lax.fori_loop(0, CH, fill_zero, 0)

    def zero_acc(k, carry):
        pltpu.sync_copy(buf0, acc_sh.at[pl.ds(sid * RPT + k * CH, CH)])
        return carry

    lax.fori_loop(0, RPT // CH, zero_acc, 0)
    pltpu.sync_copy(e_hbm.at[1, pl.ds(base, EPW)], isrc.at[pl.ds(0, EPW)])
    plsc.subcore_barrier()

    @pl.when(wid < XTRA)
    def _():
        xoff = (NW * CPW + wid) * CH
        pltpu.sync_copy(
            e_hbm.at[1, pl.ds(xoff, CH)], isrc.at[pl.ds(EPW, CH)]
        )
        pltpu.sync_copy(e_hbm.at[0, pl.ds(xoff, CH)], idst.at[0])
        pltpu.async_copy(y_hbm.at[isrc.at[pl.ds(EPW, CH)]], buf0, sem0)
        pltpu.make_async_copy(
            y_hbm.at[isrc.at[pl.ds(EPW, CH)]], buf0, sem0
        ).wait()
        pltpu.sync_copy(buf0, acc_sh.at[idst.at[0]], add=True)

    pltpu.async_copy(e_hbm.at[0, pl.ds(base, CH)], idst.at[0], semd0)
    pltpu.async_copy(e_hbm.at[0, pl.ds(base + CH, CH)], idst.at[1], semd1)
    pltpu.async_copy(y_hbm.at[isrc.at[pl.ds(0, CH)]], buf0, sem0)
    pltpu.async_copy(y_hbm.at[isrc.at[pl.ds(CH, CH)]], buf1, sem1)

    def pair(g, carry):
        c0 = 2 * g
        pltpu.make_async_copy(
            y_hbm.at[isrc.at[pl.ds(c0 * CH, CH)]], buf0, sem0
        ).wait()
        pltpu.make_async_copy(
            e_hbm.at[0, pl.ds(base + c0 * CH, CH)], idst.at[0], semd0
        ).wait()
        pltpu.sync_copy(buf0, acc_sh.at[idst.at[0]], add=True)

        @pl.when(g < CPW // 2 - 1)
        def _():
            pltpu.async_copy(
                e_hbm.at[0, pl.ds(base + (c0 + 2) * CH, CH)],
                idst.at[0], semd0,
            )
            pltpu.async_copy(
                y_hbm.at[isrc.at[pl.ds((c0 + 2) * CH, CH)]], buf0, sem0
            )

        pltpu.make_async_copy(
            y_hbm.at[isrc.at[pl.ds((c0 + 1) * CH, CH)]], buf1, sem1
        ).wait()
        pltpu.make_async_copy(
            e_hbm.at[0, pl.ds(base + (c0 + 1) * CH, CH)], idst.at[1], semd1
        ).wait()
        pltpu.sync_copy(buf1, acc_sh.at[idst.at[1]], add=True)

        @pl.when(g < CPW // 2 - 1)
        def _():
            pltpu.async_copy(
                e_hbm.at[0, pl.ds(base + (c0 + 3) * CH, CH)],
                idst.at[1], semd1,
            )
            pltpu.async_copy(
                y_hbm.at[isrc.at[pl.ds((c0 + 3) * CH, CH)]], buf1, sem1
            )

        return carry

    lax.fori_loop(0, CPW // 2, pair, 0)
    plsc.subcore_barrier()
    pltpu.sync_copy(
        acc_sh.at[pl.ds(sid * RPT, RPT)],
        acc_hbm.at[cid, pl.ds(sid * RPT, RPT)],
    )


def _t1_body(da_ref, db_ref, x_ref, dis_ref, dis2_ref, y_ref):
    deg = da_ref[...] + db_ref[...] + 1.0
    dis = lax.rsqrt(deg)
    dis_ref[...] = dis
    dis2_ref[...] = dis * dis
    y_ref[: N_NODES, :] = dis[: N_NODES, :] * x_ref[...]
    y_ref[N_NODES :, :] = jnp.zeros((NP - N_NODES, D), jnp.float32)


_t1 = pl.pallas_call(
    _t1_body,
    out_shape=(
        jax.ShapeDtypeStruct((NP, 1), jnp.float32),
        jax.ShapeDtypeStruct((NP, 1), jnp.float32),
        jax.ShapeDtypeStruct((NP, D), jnp.float32),
    ),
)


def _combine_body(s_ref, acc_ref, y_ref, o_ref):
    o_ref[...] = s_ref[...] * (acc_ref[0] + acc_ref[1] + y_ref[...])


_combine = pl.pallas_call(
    _combine_body,
    out_shape=jax.ShapeDtypeStruct((NP, D), jnp.float32),
)


def _combine_final_body(s_ref, acc_ref, y_ref, o_ref):
    n = N_NODES
    o_ref[...] = s_ref[:n, :] * (
        acc_ref[0, :n, :] + acc_ref[1, :n, :] + y_ref[:n, :]
    )


_combine_final = pl.pallas_call(
    _combine_final_body,
    out_shape=jax.ShapeDtypeStruct((N_NODES, D), jnp.float32),
)


def kernel(edge_index, x):
    ei = edge_index.astype(jnp.int32)
    degp = _deg_kernel(ei)
    da = degp[0].reshape(NP, 1)
    db = degp[1].reshape(NP, 1)
    dis, dis2, y1 = _t1(da, db, x)
    acc1 = _spmm_kernel(y1, ei)
    y2 = _combine(dis2, acc1, y1)
    acc2 = _spmm_kernel(y2, ei)
    return _combine_final(dis, acc2, y2)


# speedup vs baseline: 1.0451x; 1.0096x over previous
import functools

import jax
import jax.numpy as jnp
from jax import lax
from jax.experimental import pallas as pl
from jax.experimental.pallas import tpu as pltpu
from jax.experimental.pallas import tpu_sc as plsc

N_NODES = 10000
D = 128
NP = 10240
NC = 2
NS = 16
NW = NC * NS
CH = 128
CPW = 78
XTRA = 4
EPW = CPW * CH
RPT = NP // NS

_mesh = plsc.VectorSubcoreMesh(
    core_axis_name="c", subcore_axis_name="s", num_cores=NC, num_subcores=NS
)


@functools.partial(
    pl.kernel,
    out_type=jax.ShapeDtypeStruct((NC, NP), jnp.float32),
    mesh=_mesh,
    scratch_types=[
        pltpu.VMEM((6, CH), jnp.int32),
        pltpu.VMEM((CH,), jnp.float32),
        pltpu.VMEM((RPT,), jnp.float32),
        pltpu.VMEM_SHARED((NP,), jnp.float32),
        pltpu.SemaphoreType.DMA,
        pltpu.SemaphoreType.DMA,
        pltpu.SemaphoreType.DMA,
        pltpu.SemaphoreType.DMA,
        pltpu.SemaphoreType.DMA,
        pltpu.SemaphoreType.DMA,
    ],
)
def _deg_kernel(e_hbm, degp_hbm, idx, ones_v, z_v, deg_sh,
                semd0, semd1, semd2, semd3, semd4, semd5):
    cid = lax.axis_index("c")
    sid = lax.axis_index("s")
    wid = sid * NC + cid
    base = wid * EPW
    semd = (semd0, semd1, semd2, semd3, semd4, semd5)

    def fill_zero(i, carry):
        z_v[pl.ds(i * 16, 16)] = jnp.zeros((16,), jnp.float32)
        return carry

    lax.fori_loop(0, RPT // 16, fill_zero, 0)
    for j in range(CH // 16):
        ones_v[pl.ds(j * 16, 16)] = jnp.ones((16,), jnp.float32)
    pltpu.sync_copy(z_v, deg_sh.at[pl.ds(sid * RPT, RPT)])
    plsc.subcore_barrier()

    for j in range(6):
        pltpu.async_copy(
            e_hbm.at[1, pl.ds(base + j * CH, CH)], idx.at[j], semd[j]
        )

    def hexa(h, carry):
        c0 = 6 * h
        for j in range(6):
            pltpu.make_async_copy(
                e_hbm.at[1, pl.ds(base + (c0 + j) * CH, CH)],
                idx.at[j], semd[j],
            ).wait()
            pltpu.sync_copy(ones_v, deg_sh.at[idx.at[j]], add=True)

            @pl.when(c0 + j + 6 < CPW)
            def _():
                pltpu.async_copy(
                    e_hbm.at[1, pl.ds(base + (c0 + j + 6) * CH, CH)],
                    idx.at[j], semd[j],
                )

        return carry

    lax.fori_loop(0, CPW // 6, hexa, 0)

    @pl.when(wid < XTRA)
    def _():
        pltpu.sync_copy(
            e_hbm.at[1, pl.ds((NW * CPW + wid) * CH, CH)], idx.at[0]
        )
        pltpu.sync_copy(ones_v, deg_sh.at[idx.at[0]], add=True)

    plsc.subcore_barrier()
    pltpu.sync_copy(
        deg_sh.at[pl.ds(sid * RPT, RPT)],
        degp_hbm.at[cid, pl.ds(sid * RPT, RPT)],
    )


@functools.partial(
    pl.kernel,
    out_type=jax.ShapeDtypeStruct((NC, NP, D), jnp.float32),
    mesh=_mesh,
    scratch_types=[
        pltpu.VMEM((EPW + CH, ), jnp.int32),
        pltpu.VMEM((2, CH), jnp.int32),
        pltpu.VMEM((CH, D), jnp.float32),
        pltpu.VMEM((CH, D), jnp.float32),
        pltpu.VMEM_SHARED((NP, D), jnp.float32),
        pltpu.SemaphoreType.DMA,
        pltpu.SemaphoreType.DMA,
        pltpu.SemaphoreType.DMA,
        pltpu.SemaphoreType.DMA,
    ],
)
def _spmm_kernel(y_hbm, e_hbm, acc_hbm, isrc, idst, buf0, buf1,
                 acc_sh, sem0, sem1, semd0, semd1):
    cid = lax.axis_index("c")
    sid = lax.axis_index("s")
    wid = sid * NC + cid
    base = wid * EPW

    zeros16 = jnp.zeros((16,), jnp.float32)

    def fill_zero(i, carry):
        for j in range(D // 16):
            buf0[i, pl.ds(j * 16, 16)] = zeros16
        return carry

    lax.fori_loop(0, CH, fill_zero, 0)

    def zero_acc(k, carry):
        pltpu.sync_copy(buf0, acc_sh.at[pl.ds(sid * RPT + k * CH, CH)])
        return carry

    lax.fori_loop(0, RPT // CH, zero_acc, 0)
    pltpu.sync_copy(e_hbm.at[1, pl.ds(base, EPW)], isrc.at[pl.ds(0, EPW)])
    plsc.subcore_barrier()

    @pl.when(wid < XTRA)
    def _():
        xoff = (NW * CPW + wid) * CH
        pltpu.sync_copy(
            e_hbm.at[1, pl.ds(xoff, CH)], isrc.at[pl.ds(EPW, CH)]
        )
        pltpu.sync_copy(e_hbm.at[0, pl.ds(xoff, CH)], idst.at[0])
        pltpu.async_copy(y_hbm.at[isrc.at[pl.ds(EPW, CH)]], buf0, sem0)
        pltpu.make_async_copy(
            y_hbm.at[isrc.at[pl.ds(EPW, CH)]], buf0, sem0
        ).wait()
        pltpu.sync_copy(buf0, acc_sh.at[idst.at[0]], add=True)

    pltpu.async_copy(e_hbm.at[0, pl.ds(base, CH)], idst.at[0], semd0)
    pltpu.async_copy(e_hbm.at[0, pl.ds(base + CH, CH)], idst.at[1], semd1)
    pltpu.async_copy(y_hbm.at[isrc.at[pl.ds(0, CH)]], buf0, sem0)
    pltpu.async_copy(y_hbm.at[isrc.at[pl.ds(CH, CH)]], buf1, sem1)

    def pair(g, carry):
        c0 = 2 * g
        pltpu.make_async_copy(
            y_hbm.at[isrc.at[pl.ds(c0 * CH, CH)]], buf0, sem0
        ).wait()
        pltpu.make_async_copy(
            e_hbm.at[0, pl.ds(base + c0 * CH, CH)], idst.at[0], semd0
        ).wait()
        pltpu.sync_copy(buf0, acc_sh.at[idst.at[0]], add=True)

        @pl.when(g < CPW // 2 - 1)
        def _():
            pltpu.async_copy(
                e_hbm.at[0, pl.ds(base + (c0 + 2) * CH, CH)],
                idst.at[0], semd0,
            )
            pltpu.async_copy(
                y_hbm.at[isrc.at[pl.ds((c0 + 2) * CH, CH)]], buf0, sem0
            )

        pltpu.make_async_copy(
            y_hbm.at[isrc.at[pl.ds((c0 + 1) * CH, CH)]], buf1, sem1
        ).wait()
        pltpu.make_async_copy(
            e_hbm.at[0, pl.ds(base + (c0 + 1) * CH, CH)], idst.at[1], semd1
        ).wait()
        pltpu.sync_copy(buf1, acc_sh.at[idst.at[1]], add=True)

        @pl.when(g < CPW // 2 - 1)
        def _():
            pltpu.async_copy(
                e_hbm.at[0, pl.ds(base + (c0 + 3) * CH, CH)],
                idst.at[1], semd1,
            )
            pltpu.async_copy(
                y_hbm.at[isrc.at[pl.ds((c0 + 3) * CH, CH)]], buf1, sem1
            )

        return carry

    lax.fori_loop(0, CPW // 2, pair, 0)
    plsc.subcore_barrier()
    pltpu.sync_copy(
        acc_sh.at[pl.ds(sid * RPT, RPT)],
        acc_hbm.at[cid, pl.ds(sid * RPT, RPT)],
    )


def _t1_body(da_ref, db_ref, x_ref, dis_ref, dis2_ref, y_ref):
    deg = da_ref[...] + db_ref[...] + 1.0
    dis = lax.rsqrt(deg)
    dis_ref[...] = dis
    dis2_ref[...] = dis * dis
    y_ref[: N_NODES, :] = dis[: N_NODES, :] * x_ref[...]
    y_ref[N_NODES :, :] = jnp.zeros((NP - N_NODES, D), jnp.float32)


_t1 = pl.pallas_call(
    _t1_body,
    out_shape=(
        jax.ShapeDtypeStruct((NP, 1), jnp.float32),
        jax.ShapeDtypeStruct((NP, 1), jnp.float32),
        jax.ShapeDtypeStruct((NP, D), jnp.float32),
    ),
)


def _combine_body(s_ref, acc_ref, y_ref, o_ref):
    o_ref[...] = s_ref[...] * (acc_ref[0] + acc_ref[1] + y_ref[...])


_combine = pl.pallas_call(
    _combine_body,
    out_shape=jax.ShapeDtypeStruct((NP, D), jnp.float32),
)


def _combine_final_body(s_ref, acc_ref, y_ref, o_ref):
    n = N_NODES
    o_ref[...] = s_ref[:n, :] * (
        acc_ref[0, :n, :] + acc_ref[1, :n, :] + y_ref[:n, :]
    )


_combine_final = pl.pallas_call(
    _combine_final_body,
    out_shape=jax.ShapeDtypeStruct((N_NODES, D), jnp.float32),
)


def kernel(edge_index, x):
    ei = edge_index.astype(jnp.int32)
    degp = _deg_kernel(ei)
    da = degp[0].reshape(NP, 1)
    db = degp[1].reshape(NP, 1)
    dis, dis2, y1 = _t1(da, db, x)
    acc1 = _spmm_kernel(y1, ei)
    y2 = _combine(dis2, acc1, y1)
    acc2 = _spmm_kernel(y2, ei)
    return _combine_final(dis, acc2, y2)
